# Initial kernel scaffold; baseline (speedup 1.0000x reference)
import jax
import jax.numpy as jnp
from jax.experimental import pallas as pl


def kernel(positive_head, positive_tail, positive_relation, negtive_head, negtive_tail, negtive_relation, attn_mask, rel_table, tail_table):
    raise NotImplementedError("write your pallas kernel here")


# speedup vs baseline: 1.2450x; 1.2450x over previous
import functools

import jax
import jax.numpy as jnp
from jax import lax
from jax.experimental import pallas as pl
from jax.experimental.pallas import tpu as pltpu
from jax.experimental.pallas import tpu_sc as plsc

DIM = 64
LANES = 16
CHUNK = 128
MARGIN = 1.0


def _build(n_rows):
    info = plsc.get_sparse_core_info()
    nc, ns = info.num_cores, info.num_subcores
    n_workers = nc * ns
    rows_per_w = n_rows // n_workers
    n_chunks = rows_per_w // CHUNK
    n_groups = CHUNK // LANES
    mesh = plsc.VectorSubcoreMesh(core_axis_name="c", subcore_axis_name="s")

    @functools.partial(
        pl.kernel,
        mesh=mesh,
        compiler_params=pltpu.CompilerParams(
            needs_layout_passes=False, use_tc_tiling_on_sc=False),
        out_type=(
            jax.ShapeDtypeStruct((n_rows,), jnp.float32),
            jax.ShapeDtypeStruct((n_rows, DIM), jnp.float32),
        ),
        scratch_types=[
            pltpu.VMEM((CHUNK,), jnp.int32),
            pltpu.VMEM((CHUNK,), jnp.int32),
            pltpu.VMEM((CHUNK,), jnp.int32),
            pltpu.VMEM((CHUNK,), jnp.int32),
            pltpu.VMEM((CHUNK,), jnp.float32),
            pltpu.VMEM((CHUNK, DIM), jnp.float32),
            pltpu.VMEM((CHUNK, DIM), jnp.float32),
            pltpu.VMEM((CHUNK, DIM), jnp.float32),
            pltpu.VMEM((CHUNK, DIM), jnp.float32),
            pltpu.VMEM((CHUNK, DIM), jnp.float32),
            pltpu.VMEM((CHUNK, DIM), jnp.float32),
            pltpu.VMEM((CHUNK,), jnp.float32),
            pltpu.VMEM((LANES * LANES,), jnp.float32),
            pltpu.SemaphoreType.DMA,
        ],
    )
    def k(ph_hbm, nh_hbm, pti_hbm, nti_hbm, pri_hbm, nri_hbm, mask_hbm,
          rel_hbm, tail_hbm, loss_hbm, relout_hbm,
          pt_i, nt_i, pr_i, nr_i, mask_v, ph_v, nh_v, pt_v, nt_v, pr_v, nr_v,
          loss_v, stage_v, sem):
        wid = lax.axis_index("s") * nc + lax.axis_index("c")
        base0 = wid * rows_per_w
        iota = lax.iota(jnp.int32, LANES)

        def chunk_body(ci, carry):
            base = base0 + ci * CHUNK
            sl = pl.ds(base, CHUNK)
            pltpu.sync_copy(pti_hbm.at[sl], pt_i)
            pltpu.sync_copy(nti_hbm.at[sl], nt_i)
            pltpu.sync_copy(pri_hbm.at[sl], pr_i)
            pltpu.sync_copy(nri_hbm.at[sl], nr_i)
            c1 = pltpu.async_copy(tail_hbm.at[pt_i], pt_v, sem)
            c2 = pltpu.async_copy(tail_hbm.at[nt_i], nt_v, sem)
            c3 = pltpu.async_copy(rel_hbm.at[pr_i], pr_v, sem)
            c4 = pltpu.async_copy(rel_hbm.at[nr_i], nr_v, sem)
            pltpu.sync_copy(ph_hbm.at[sl], ph_v)
            pltpu.sync_copy(nh_hbm.at[sl], nh_v)
            pltpu.sync_copy(mask_hbm.at[sl], mask_v)
            c1.wait()
            c2.wait()
            c3.wait()
            c4.wait()

            def group_body(g, carry2):
                for rr in range(LANES):
                    r = g * LANES + rr
                    acc0 = jnp.zeros((LANES,), jnp.float32)
                    acc1 = jnp.zeros((LANES,), jnp.float32)
                    for j in range(DIM // LANES):
                        js = pl.ds(j * LANES, LANES)
                        pterm = jnp.abs(ph_v[r, js] + pr_v[r, js]
                                        - pt_v[r, js])
                        nterm = jnp.abs(nh_v[r, js] + nr_v[r, js]
                                        - nt_v[r, js])
                        if j % 2 == 0:
                            acc0 = acc0 + (pterm - nterm)
                        else:
                            acc1 = acc1 + (pterm - nterm)
                    stage_v[pl.ds(rr * LANES, LANES)] = acc0 + acc1
                tot = jnp.zeros((LANES,), jnp.float32)
                for j in range(LANES):
                    tot = tot + plsc.load_gather(stage_v, [iota * LANES + j])
                gsl = pl.ds(g * LANES, LANES)
                loss_v[gsl] = jnp.maximum(mask_v[gsl] * tot + MARGIN, 0.0)
                return carry2

            lax.fori_loop(0, n_groups, group_body, 0)
            pltpu.sync_copy(loss_v, loss_hbm.at[sl])
            pltpu.sync_copy(pr_v, relout_hbm.at[sl])
            return carry

        lax.fori_loop(0, n_chunks, chunk_body, 0)

    return k


def kernel(positive_head, positive_tail, positive_relation, negtive_head,
           negtive_tail, negtive_relation, attn_mask, rel_table, tail_table):
    b, l, d = positive_head.shape
    n = b * l
    ph = positive_head.reshape(n, d)
    nh = negtive_head.reshape(n, d)
    pti = positive_tail.reshape(n).astype(jnp.int32)
    nti = negtive_tail.reshape(n).astype(jnp.int32)
    pri = positive_relation.reshape(n).astype(jnp.int32)
    nri = negtive_relation.reshape(n).astype(jnp.int32)
    mask = attn_mask.reshape(n).astype(jnp.float32)
    loss, rel_rows = _build(n)(ph, nh, pti, nti, pri, nri, mask,
                               rel_table, tail_table)
    return loss.reshape(n, 1), rel_rows.reshape(b, l, d)


# speedup vs baseline: 1.4058x; 1.1291x over previous
import functools

import jax
import jax.numpy as jnp
from jax import lax
from jax.experimental import pallas as pl
from jax.experimental.pallas import tpu as pltpu
from jax.experimental.pallas import tpu_sc as plsc

DIM = 64
LANES = 16
CHUNK = 128
MARGIN = 1.0


def _build(n_rows, n_rel):
    info = plsc.get_sparse_core_info()
    nc, ns = info.num_cores, info.num_subcores
    n_workers = nc * ns
    rows_per_w = n_rows // n_workers
    n_chunks = rows_per_w // CHUNK
    n_groups = CHUNK // LANES
    mesh = plsc.VectorSubcoreMesh(core_axis_name="c", subcore_axis_name="s")

    @functools.partial(
        pl.kernel,
        mesh=mesh,
        compiler_params=pltpu.CompilerParams(needs_layout_passes=False),
        out_type=(
            jax.ShapeDtypeStruct((n_rows,), jnp.float32),
            jax.ShapeDtypeStruct((n_rows, DIM), jnp.float32),
        ),
        scratch_types=[
            pltpu.VMEM((CHUNK,), jnp.int32),
            pltpu.VMEM((CHUNK,), jnp.int32),
            pltpu.VMEM((CHUNK,), jnp.int32),
            pltpu.VMEM((CHUNK,), jnp.int32),
            pltpu.VMEM((CHUNK,), jnp.float32),
            pltpu.VMEM((CHUNK, DIM), jnp.float32),
            pltpu.VMEM((CHUNK, DIM), jnp.float32),
            pltpu.VMEM((CHUNK, DIM), jnp.float32),
            pltpu.VMEM((CHUNK, DIM), jnp.float32),
            pltpu.VMEM((CHUNK, DIM), jnp.float32),
            pltpu.VMEM((CHUNK, DIM), jnp.float32),
            pltpu.VMEM((CHUNK,), jnp.float32),
            pltpu.VMEM((LANES * LANES,), jnp.float32),
            pltpu.SemaphoreType.DMA,
            pltpu.SemaphoreType.DMA,
        ],
    )
    def k(ph_hbm, nh_hbm, pti_hbm, nti_hbm, pri_hbm, nri_hbm, mask_hbm,
          rel_hbm, tail_hbm, loss_hbm, relout_hbm,
          pt_i, nt_i, pr_i, nr_i, mask_v, ph_v, nh_v, pt_v, nt_v,
          pr_v, nr_v, loss_v, stage_v, sem, sem2):
        wid = lax.axis_index("s") * nc + lax.axis_index("c")
        base0 = wid * rows_per_w
        iota = lax.iota(jnp.int32, LANES)

        def chunk_body(ci, carry):
            base = base0 + ci * CHUNK
            sl = pl.ds(base, CHUNK)
            pltpu.sync_copy(pti_hbm.at[sl], pt_i)
            pltpu.sync_copy(nti_hbm.at[sl], nt_i)
            pltpu.sync_copy(pri_hbm.at[sl], pr_i)
            pltpu.sync_copy(nri_hbm.at[sl], nr_i)
            pltpu.sync_copy(mask_hbm.at[sl], mask_v)
            pltpu.sync_copy(ph_hbm.at[sl], ph_v)
            pltpu.sync_copy(nh_hbm.at[sl], nh_v)

            def fire(g, carry2):
                ptv = pt_i[pl.ds(g * LANES, LANES)]
                ntv = nt_i[pl.ds(g * LANES, LANES)]
                prv = pr_i[pl.ds(g * LANES, LANES)]
                nrv = nr_i[pl.ds(g * LANES, LANES)]
                for rr in range(LANES):
                    r = g * LANES + rr
                    pltpu.async_copy(tail_hbm.at[pl.ds(ptv[rr], 1)],
                                     pt_v.at[pl.ds(r, 1)], sem)
                    pltpu.async_copy(tail_hbm.at[pl.ds(ntv[rr], 1)],
                                     nt_v.at[pl.ds(r, 1)], sem)
                    pltpu.async_copy(rel_hbm.at[pl.ds(prv[rr], 1)],
                                     pr_v.at[pl.ds(r, 1)], sem)
                    pltpu.async_copy(rel_hbm.at[pl.ds(nrv[rr], 1)],
                                     nr_v.at[pl.ds(r, 1)], sem)
                return carry2

            lax.fori_loop(0, n_groups, fire, 0)

            def drain(i, carry2):
                pltpu.make_async_copy(tail_hbm.at[pl.ds(0, 1)],
                                      pt_v.at[pl.ds(0, 1)], sem).wait()
                return carry2

            lax.fori_loop(0, 4 * CHUNK, drain, 0)

            def comp(g, carry2):
                for rr in range(LANES):
                    r = g * LANES + rr
                    acc0 = jnp.zeros((LANES,), jnp.float32)
                    acc1 = jnp.zeros((LANES,), jnp.float32)
                    for j in range(DIM // LANES):
                        js = pl.ds(j * LANES, LANES)
                        pterm = jnp.abs(ph_v[r, js] + pr_v[r, js]
                                        - pt_v[r, js])
                        nterm = jnp.abs(nh_v[r, js] + nr_v[r, js]
                                        - nt_v[r, js])
                        if j % 2 == 0:
                            acc0 = acc0 + (pterm - nterm)
                        else:
                            acc1 = acc1 + (pterm - nterm)
                    stage_v[pl.ds(rr * LANES, LANES)] = acc0 + acc1
                tot = jnp.zeros((LANES,), jnp.float32)
                for j in range(LANES):
                    tot = tot + plsc.load_gather(stage_v, [iota * LANES + j])
                gsl = pl.ds(g * LANES, LANES)
                loss_v[gsl] = jnp.maximum(mask_v[gsl] * tot + MARGIN, 0.0)
                return carry2

            lax.fori_loop(0, n_groups, comp, 0)
            pltpu.sync_copy(loss_v, loss_hbm.at[sl])
            pltpu.sync_copy(pr_v, relout_hbm.at[sl])
            return carry

        lax.fori_loop(0, n_chunks, chunk_body, 0)

    return k


def kernel(positive_head, positive_tail, positive_relation, negtive_head,
           negtive_tail, negtive_relation, attn_mask, rel_table, tail_table):
    b, l, d = positive_head.shape
    n = b * l
    ph = positive_head.reshape(n, d)
    nh = negtive_head.reshape(n, d)
    pti = positive_tail.reshape(n).astype(jnp.int32)
    nti = negtive_tail.reshape(n).astype(jnp.int32)
    pri = positive_relation.reshape(n).astype(jnp.int32)
    nri = negtive_relation.reshape(n).astype(jnp.int32)
    mask = attn_mask.reshape(n).astype(jnp.float32)
    loss, rel_rows = _build(n, rel_table.shape[0])(
        ph, nh, pti, nti, pri, nri, mask, rel_table, tail_table)
    return loss.reshape(n, 1), rel_rows.reshape(b, l, d)


# speedup vs baseline: 1.5118x; 1.0754x over previous
import functools

import jax
import jax.numpy as jnp
from jax import lax
from jax.experimental import pallas as pl
from jax.experimental.pallas import tpu as pltpu
from jax.experimental.pallas import tpu_sc as plsc

DIM = 64
LANES = 16
CHUNK = 128
MARGIN = 1.0


def _build(n_rows):
    info = plsc.get_sparse_core_info()
    nc, ns = info.num_cores, info.num_subcores
    n_workers = nc * ns
    rows_per_w = n_rows // n_workers
    n_chunks = rows_per_w // CHUNK
    n_groups = CHUNK // LANES
    mesh = plsc.VectorSubcoreMesh(core_axis_name="c", subcore_axis_name="s")

    @functools.partial(
        pl.kernel,
        mesh=mesh,
        compiler_params=pltpu.CompilerParams(needs_layout_passes=False),
        out_type=(
            jax.ShapeDtypeStruct((n_rows,), jnp.float32),
            jax.ShapeDtypeStruct((n_rows, DIM), jnp.float32),
        ),
        scratch_types=[
            pltpu.VMEM((2, CHUNK), jnp.int32),
            pltpu.VMEM((2, CHUNK), jnp.int32),
            pltpu.VMEM((2, CHUNK), jnp.int32),
            pltpu.VMEM((2, CHUNK), jnp.int32),
            pltpu.VMEM((2, CHUNK), jnp.float32),
            pltpu.VMEM((CHUNK, DIM), jnp.float32),
            pltpu.VMEM((CHUNK, DIM), jnp.float32),
            pltpu.VMEM((CHUNK, DIM), jnp.float32),
            pltpu.VMEM((CHUNK, DIM), jnp.float32),
            pltpu.VMEM((CHUNK, DIM), jnp.float32),
            pltpu.VMEM((CHUNK, DIM), jnp.float32),
            pltpu.VMEM((CHUNK,), jnp.float32),
            pltpu.VMEM((LANES * LANES,), jnp.float32),
            pltpu.SemaphoreType.DMA,
            pltpu.SemaphoreType.DMA,
            pltpu.SemaphoreType.DMA,
            pltpu.SemaphoreType.DMA,
        ],
    )
    def k(ph_hbm, nh_hbm, pti_hbm, nti_hbm, pri_hbm, nri_hbm, mask_hbm,
          rel_hbm, tail_hbm, loss_hbm, relout_hbm,
          pti2, nti2, pri2, nri2, mask2, ph_v, nh_v, pt_v, nt_v, pr_v, nr_v,
          loss_v, stage_v, sem_in, sem_heads, sem_rows, sem_out):
        wid = lax.axis_index("s") * nc + lax.axis_index("c")
        base0 = wid * rows_per_w
        iota = lax.iota(jnp.int32, LANES)

        def fire_idx(ci, b):
            sl = pl.ds(base0 + ci * CHUNK, CHUNK)
            pltpu.async_copy(pti_hbm.at[sl], pti2.at[b], sem_in)
            pltpu.async_copy(nti_hbm.at[sl], nti2.at[b], sem_in)
            pltpu.async_copy(pri_hbm.at[sl], pri2.at[b], sem_in)
            pltpu.async_copy(nri_hbm.at[sl], nri2.at[b], sem_in)
            pltpu.async_copy(mask_hbm.at[sl], mask2.at[b], sem_in)

        def drain_idx():
            for _ in range(4):
                pltpu.make_async_copy(pti_hbm.at[pl.ds(0, CHUNK)],
                                      pti2.at[0], sem_in).wait()
            pltpu.make_async_copy(mask_hbm.at[pl.ds(0, CHUNK)],
                                  mask2.at[0], sem_in).wait()

        def fire_heads(ci):
            sl = pl.ds(base0 + ci * CHUNK, CHUNK)
            pltpu.async_copy(ph_hbm.at[sl], ph_v, sem_heads)
            pltpu.async_copy(nh_hbm.at[sl], nh_v, sem_heads)

        def drain_heads():
            pltpu.make_async_copy(ph_hbm.at[pl.ds(0, CHUNK)],
                                  ph_v, sem_heads).wait()
            pltpu.make_async_copy(ph_hbm.at[pl.ds(0, CHUNK)],
                                  nh_v, sem_heads).wait()

        def fire_group(b, g):
            gsl = pl.ds(g * LANES, LANES)
            ptv = pti2[b, gsl]
            ntv = nti2[b, gsl]
            prv = pri2[b, gsl]
            nrv = nri2[b, gsl]
            for rr in range(LANES):
                r = g * LANES + rr
                pltpu.async_copy(tail_hbm.at[pl.ds(ptv[rr], 1)],
                                 pt_v.at[pl.ds(r, 1)], sem_rows)
                pltpu.async_copy(tail_hbm.at[pl.ds(ntv[rr], 1)],
                                 nt_v.at[pl.ds(r, 1)], sem_rows)
                pltpu.async_copy(rel_hbm.at[pl.ds(prv[rr], 1)],
                                 pr_v.at[pl.ds(r, 1)], sem_rows)
                pltpu.async_copy(rel_hbm.at[pl.ds(nrv[rr], 1)],
                                 nr_v.at[pl.ds(r, 1)], sem_rows)

        def drain_rows_quarter():
            def d(i, c):
                pltpu.make_async_copy(tail_hbm.at[pl.ds(0, 1)],
                                      pt_v.at[pl.ds(0, 1)], sem_rows).wait()
                return c

            lax.fori_loop(0, 4 * 2 * LANES, d, 0)

        def drain_outs():
            pltpu.make_async_copy(loss_hbm.at[pl.ds(0, CHUNK)],
                                  loss_v, sem_out).wait()
            pltpu.make_async_copy(relout_hbm.at[pl.ds(0, CHUNK)],
                                  pr_v, sem_out).wait()

        def chunk_body(ci, carry):
            b = lax.rem(ci, 2)
            sl = pl.ds(base0 + ci * CHUNK, CHUNK)

            @pl.when(ci > 0)
            def _():
                drain_outs()

            drain_idx()
            fire_heads(ci)

            @pl.when(ci + 1 < n_chunks)
            def _():
                fire_idx(ci + 1, 1 - b)

            def fire4(g, c):
                fire_group(b, g)
                return c

            lax.fori_loop(0, n_groups // 2, fire4, 0)
            drain_heads()

            def grand(gg, c):
                @pl.when(lax.rem(gg, 2) == 0)
                def _():
                    drain_rows_quarter()

                @pl.when(gg < n_groups // 2)
                def _():
                    fire_group(b, gg + n_groups // 2)

                gsl = pl.ds(gg * LANES, LANES)
                for rr in range(LANES):
                    r = gg * LANES + rr
                    acc0 = jnp.zeros((LANES,), jnp.float32)
                    acc1 = jnp.zeros((LANES,), jnp.float32)
                    for j in range(DIM // LANES):
                        js = pl.ds(j * LANES, LANES)
                        pterm = jnp.abs(ph_v[r, js] + pr_v[r, js]
                                        - pt_v[r, js])
                        nterm = jnp.abs(nh_v[r, js] + nr_v[r, js]
                                        - nt_v[r, js])
                        if j % 2 == 0:
                            acc0 = acc0 + (pterm - nterm)
                        else:
                            acc1 = acc1 + (pterm - nterm)
                    stage_v[pl.ds(rr * LANES, LANES)] = acc0 + acc1
                tot = jnp.zeros((LANES,), jnp.float32)
                for j in range(LANES):
                    tot = tot + plsc.load_gather(stage_v, [iota * LANES + j])
                loss_v[gsl] = jnp.maximum(mask2[b, gsl] * tot + MARGIN, 0.0)
                return c

            lax.fori_loop(0, n_groups, grand, 0)
            pltpu.async_copy(loss_v, loss_hbm.at[sl], sem_out)
            pltpu.async_copy(pr_v, relout_hbm.at[sl], sem_out)
            return carry

        fire_idx(0, 0)
        lax.fori_loop(0, n_chunks, chunk_body, 0)
        drain_outs()

    return k


def kernel(positive_head, positive_tail, positive_relation, negtive_head,
           negtive_tail, negtive_relation, attn_mask, rel_table, tail_table):
    b, l, d = positive_head.shape
    n = b * l
    ph = positive_head.reshape(n, d)
    nh = negtive_head.reshape(n, d)
    pti = positive_tail.reshape(n).astype(jnp.int32)
    nti = negtive_tail.reshape(n).astype(jnp.int32)
    pri = positive_relation.reshape(n).astype(jnp.int32)
    nri = negtive_relation.reshape(n).astype(jnp.int32)
    mask = attn_mask.reshape(n).astype(jnp.float32)
    loss, rel_rows = _build(n)(
        ph, nh, pti, nti, pri, nri, mask, rel_table, tail_table)
    return loss.reshape(n, 1), rel_rows.reshape(b, l, d)
